# baseline (speedup 1.0000x reference)
import jax
import jax.numpy as jnp
from jax import lax
from jax.experimental import pallas as pl
from jax.experimental.pallas import tpu as pltpu
from jax.experimental.pallas import tpu_sc as plsc

B = 16384
D = 64
NC = 2
NS = 16
L = 16
NW = NC * NS
BPW = B // NW
IPW = B // NW
TILE_W = 128
NSLOT = 4
RSW = 5 * 128


def _sread(flat, i):
    return flat[pl.ds(i, L)][0]


def _k1_body(sid_hbm, col_hbm, rs_hbm, tab_hbm, gath_hbm,
             sidf, colf, rsf, blk_v, outbox_v, sem):
    wid = lax.axis_index("s") * NC + lax.axis_index("c")
    lane = lax.iota(jnp.int32, L)

    pltpu.sync_copy(sid_hbm.at[pl.ds(wid * IPW, IPW)], sidf.at[pl.ds(0, IPW)])
    pltpu.sync_copy(col_hbm.at[pl.ds(wid * IPW, IPW)], colf.at[pl.ds(0, IPW)])
    pltpu.sync_copy(rs_hbm.at[pl.ds(wid * RSW, RSW)], rsf.at[pl.ds(0, RSW)])

    ncols = rsf[pl.ds(RSW - L, L)][L - 1]
    nbatch = (ncols + NSLOT - 1) // NSLOT

    def batch_body(q, carry):
        for s in range(NSLOT):
            col = _sread(colf, q * NSLOT + s)
            start = pl.multiple_of(col * TILE_W, TILE_W)
            pltpu.async_copy(tab_hbm.at[:, pl.ds(start, TILE_W)],
                             blk_v.at[s], sem)
        for s in range(NSLOT):
            pltpu.make_async_copy(tab_hbm.at[:, pl.ds(0, TILE_W)],
                                  blk_v.at[s], sem).wait()
        for s in range(NSLOT):
            f = q * NSLOT + s
            rs0 = _sread(rsf, f)
            rs1 = _sread(rsf, f + 1)

            def id_body(i, carry2, s=s):
                sid = _sread(sidf, i)
                jv = jnp.full((L,), lax.rem(sid, TILE_W), jnp.int32)
                sv = jnp.full((L,), s, jnp.int32)
                rowv = jnp.full((L,), i, jnp.int32)
                for k in range(D // L):
                    val = plsc.load_gather(blk_v, [sv, lane + k * L, jv])
                    plsc.store_scatter(outbox_v, [rowv, lane + k * L], val)
                return carry2

            lax.fori_loop(rs0, rs1, id_body, 0)
        return carry

    lax.fori_loop(0, nbatch, batch_body, 0)

    pltpu.sync_copy(outbox_v, gath_hbm.at[pl.ds(wid * IPW, IPW), :])


def _k2_body(dpos_hbm, rpos_hbm, dgath_hbm, rgath_hbm, out_hbm,
             didx_v, ridx_v, drows_v, rrows_v, out_v, sem):
    wid = lax.axis_index("s") * NC + lax.axis_index("c")
    base = wid * BPW
    lane = lax.iota(jnp.int32, L)

    pltpu.sync_copy(dpos_hbm.at[wid], didx_v)
    pltpu.sync_copy(rpos_hbm.at[wid], ridx_v)

    for h in range(2):
        copies = []
        for c in range(2):
            dst = pl.ds(c * 128, 128)
            copies.append(pltpu.async_copy(
                dgath_hbm.at[didx_v.at[2 * h + c]], drows_v.at[dst], sem))
            copies.append(pltpu.async_copy(
                rgath_hbm.at[ridx_v.at[2 * h + c]], rrows_v.at[dst], sem))
        for cp in copies:
            cp.wait()
        for m in range(256 // L):
            rowv = m * L + lane
            acc = jnp.zeros((L,), jnp.float32)
            for d in range(D):
                dv = jnp.full((L,), d, jnp.int32)
                a = plsc.load_gather(drows_v, [rowv, dv])
                b = plsc.load_gather(rrows_v, [rowv, dv])
                acc = acc + a * b
            out_v[pl.ds(h * 256 + m * L, L)] = acc

    pltpu.sync_copy(out_v, out_hbm.at[pl.ds(base, BPW)])


def _sched(ids):
    ids = ids.astype(jnp.int32)
    order = jnp.argsort(ids).astype(jnp.int32)
    sid = jnp.take(ids, order)
    pos = jnp.zeros((B,), jnp.int32).at[order].set(
        jnp.arange(B, dtype=jnp.int32))
    colsw = (sid // TILE_W).reshape(NW, IPW)
    first = jnp.concatenate(
        [jnp.ones((NW, 1), jnp.bool_), colsw[:, 1:] != colsw[:, :-1]], axis=1)
    fno = jnp.cumsum(first.astype(jnp.int32), axis=1) - 1
    ncols = fno[:, -1] + 1
    rows = jnp.arange(NW, dtype=jnp.int32)[:, None]
    col_list = jnp.zeros((NW, IPW), jnp.int32).at[rows, fno].set(colsw)
    iidx = jnp.broadcast_to(jnp.arange(IPW, dtype=jnp.int32), (NW, IPW))
    runstart = jnp.full((NW, RSW), IPW, jnp.int32).at[rows, fno].min(iidx)
    runstart = runstart.at[:, RSW - 1].set(ncols)
    return (sid.reshape(NW * IPW), col_list.reshape(NW * IPW),
            runstart.reshape(NW * RSW), pos)


def kernel(donor_ids, receiver_ids, donor_table, receiver_table):
    dsid, dcol, drs, dpos = _sched(donor_ids)
    rsid, rcol, rrs, rpos = _sched(receiver_ids)

    mesh = plsc.VectorSubcoreMesh(core_axis_name="c", subcore_axis_name="s",
                                  num_cores=NC, num_subcores=NS)
    params = pltpu.CompilerParams(needs_layout_passes=False)

    k1 = pl.kernel(
        _k1_body,
        out_type=jax.ShapeDtypeStruct((B, 128), jnp.float32),
        mesh=mesh,
        compiler_params=params,
        scratch_types=[
            pltpu.VMEM((IPW + L,), jnp.int32),
            pltpu.VMEM((IPW + L,), jnp.int32),
            pltpu.VMEM((RSW + L,), jnp.int32),
            pltpu.VMEM((NSLOT, D, TILE_W), jnp.float32),
            pltpu.VMEM((IPW, 128), jnp.float32),
            pltpu.SemaphoreType.DMA,
        ],
    )
    dgath = k1(dsid, dcol, drs, donor_table.T)
    rgath = k1(rsid, rcol, rrs, receiver_table.T)

    k2 = pl.kernel(
        _k2_body,
        out_type=jax.ShapeDtypeStruct((B,), jnp.float32),
        mesh=mesh,
        compiler_params=params,
        scratch_types=[
            pltpu.VMEM((4, 128), jnp.int32),
            pltpu.VMEM((4, 128), jnp.int32),
            pltpu.VMEM((256, 128), jnp.float32),
            pltpu.VMEM((256, 128), jnp.float32),
            pltpu.VMEM((BPW,), jnp.float32),
            pltpu.SemaphoreType.DMA,
        ],
    )
    return k2(dpos.reshape(NW, 4, 128), rpos.reshape(NW, 4, 128),
              dgath, rgath)


# speedup vs baseline: 1.4493x; 1.4493x over previous
import jax
import jax.numpy as jnp
from jax import lax
from jax.experimental import pallas as pl
from jax.experimental.pallas import tpu as pltpu
from jax.experimental.pallas import tpu_sc as plsc

B = 16384
D = 64
NC = 2
NS = 16
L = 16
NW = NC * NS
BPW = B // NW
IPW = B // NW
TILE_W = 128
NSLOT = 4
RSW = 5 * 128


def _sread(flat, i):
    return flat[pl.ds(i, L)][0]


def _k1_body(sid_hbm, rs_hbm, tab_hbm, gath_hbm,
             sidf, rsf, blk_v, outbox_v, sem):
    wid = lax.axis_index("s") * NC + lax.axis_index("c")
    lane = lax.iota(jnp.int32, L)

    pltpu.sync_copy(sid_hbm.at[pl.ds(wid * IPW, IPW)], sidf.at[pl.ds(0, IPW)])
    pltpu.sync_copy(rs_hbm.at[pl.ds(wid * RSW, RSW)], rsf.at[pl.ds(0, RSW)])
    sidf[pl.ds(IPW, L)] = jnp.zeros((L,), jnp.int32)

    ncols = rsf[pl.ds(RSW - L, L)][L - 1]
    nbatch = (ncols + NSLOT - 1) // NSLOT

    def batch_body(q, carry):
        runs = []
        for s in range(NSLOT):
            f = q * NSLOT + s
            rs0 = _sread(rsf, f)
            rs1 = _sread(rsf, f + 1)
            runs.append((rs0, rs1))
            col = _sread(sidf, rs0) // TILE_W
            start = pl.multiple_of(col * TILE_W, TILE_W)
            pltpu.async_copy(tab_hbm.at[:, pl.ds(start, TILE_W)],
                             blk_v.at[s], sem)
        for s in range(NSLOT):
            pltpu.make_async_copy(tab_hbm.at[:, pl.ds(0, TILE_W)],
                                  blk_v.at[s], sem).wait()
        for s in range(NSLOT):
            rs0, rs1 = runs[s]

            def id_body(i, carry2, s=s):
                sid = _sread(sidf, i)
                jv = jnp.full((L,), lax.rem(sid, TILE_W), jnp.int32)
                sv = jnp.full((L,), s, jnp.int32)
                rowv = jnp.full((L,), i, jnp.int32)
                for k in range(D // L):
                    val = plsc.load_gather(blk_v, [sv, lane + k * L, jv])
                    plsc.store_scatter(outbox_v, [rowv, lane + k * L], val)
                return carry2

            lax.fori_loop(rs0, rs1, id_body, 0)
        return carry

    lax.fori_loop(0, nbatch, batch_body, 0)

    pltpu.sync_copy(outbox_v, gath_hbm.at[pl.ds(wid * IPW, IPW), :])


def _k2_body(dpos_hbm, rpos_hbm, dgath_hbm, rgath_hbm, out_hbm,
             didx_v, ridx_v, drows_v, rrows_v, out_v, sem):
    wid = lax.axis_index("s") * NC + lax.axis_index("c")
    base = wid * BPW
    lane = lax.iota(jnp.int32, L)

    pltpu.sync_copy(dpos_hbm.at[wid], didx_v)
    pltpu.sync_copy(rpos_hbm.at[wid], ridx_v)

    for h in range(2):
        copies = []
        for c in range(2):
            dst = pl.ds(c * 128, 128)
            copies.append(pltpu.async_copy(
                dgath_hbm.at[didx_v.at[2 * h + c]], drows_v.at[dst], sem))
            copies.append(pltpu.async_copy(
                rgath_hbm.at[ridx_v.at[2 * h + c]], rrows_v.at[dst], sem))
        for cp in copies:
            cp.wait()
        for m in range(256 // L):
            rowv = m * L + lane
            acc = jnp.zeros((L,), jnp.float32)
            for d in range(D):
                dv = jnp.full((L,), d, jnp.int32)
                a = plsc.load_gather(drows_v, [rowv, dv])
                b = plsc.load_gather(rrows_v, [rowv, dv])
                acc = acc + a * b
            out_v[pl.ds(h * 256 + m * L, L)] = acc

    pltpu.sync_copy(out_v, out_hbm.at[pl.ds(base, BPW)])


def _sched(ids):
    ids = ids.astype(jnp.int32)
    order = jnp.argsort(ids).astype(jnp.int32)
    sid = jnp.take(ids, order)
    pos = jnp.argsort(order).astype(jnp.int32)
    colsw = (sid // TILE_W).reshape(NW, IPW)
    first = jnp.concatenate(
        [jnp.ones((NW, 1), jnp.bool_), colsw[:, 1:] != colsw[:, :-1]], axis=1)
    fno = jnp.cumsum(first.astype(jnp.int32), axis=1) - 1
    ncols = fno[:, -1] + 1
    frange = jnp.arange(RSW, dtype=jnp.int32)
    runstart = jnp.sum(
        fno[:, None, :] < frange[None, :, None], axis=-1, dtype=jnp.int32)
    runstart = runstart.at[:, RSW - 1].set(ncols)
    return (sid.reshape(NW * IPW), runstart.reshape(NW * RSW), pos)


def kernel(donor_ids, receiver_ids, donor_table, receiver_table):
    dsid, drs, dpos = _sched(donor_ids)
    rsid, rrs, rpos = _sched(receiver_ids)

    mesh = plsc.VectorSubcoreMesh(core_axis_name="c", subcore_axis_name="s",
                                  num_cores=NC, num_subcores=NS)
    params = pltpu.CompilerParams(needs_layout_passes=False)

    k1 = pl.kernel(
        _k1_body,
        out_type=jax.ShapeDtypeStruct((B, 128), jnp.float32),
        mesh=mesh,
        compiler_params=params,
        scratch_types=[
            pltpu.VMEM((IPW + L,), jnp.int32),
            pltpu.VMEM((RSW + L,), jnp.int32),
            pltpu.VMEM((NSLOT, D, TILE_W), jnp.float32),
            pltpu.VMEM((IPW, 128), jnp.float32),
            pltpu.SemaphoreType.DMA,
        ],
    )
    dgath = k1(dsid, drs, donor_table.T)
    rgath = k1(rsid, rrs, receiver_table.T)

    k2 = pl.kernel(
        _k2_body,
        out_type=jax.ShapeDtypeStruct((B,), jnp.float32),
        mesh=mesh,
        compiler_params=params,
        scratch_types=[
            pltpu.VMEM((4, 128), jnp.int32),
            pltpu.VMEM((4, 128), jnp.int32),
            pltpu.VMEM((256, 128), jnp.float32),
            pltpu.VMEM((256, 128), jnp.float32),
            pltpu.VMEM((BPW,), jnp.float32),
            pltpu.SemaphoreType.DMA,
        ],
    )
    return k2(dpos.reshape(NW, 4, 128), rpos.reshape(NW, 4, 128),
              dgath, rgath)


# speedup vs baseline: 1.5214x; 1.0498x over previous
import jax
import jax.numpy as jnp
from jax import lax
from jax.experimental import pallas as pl
from jax.experimental.pallas import tpu as pltpu
from jax.experimental.pallas import tpu_sc as plsc

B = 16384
D = 64
NC = 2
NS = 16
L = 16
NW = NC * NS
BPW = B // NW
IPW = B // NW
TILE_W = 128
NSLOT = 6
RSW = 5 * 128


def _sread(flat, i):
    return flat[pl.ds(i, L)][0]


def _k1_body(sid_hbm, rs_hbm, tab_hbm, gath_hbm,
             sidf, rsf, blk_v, outbox_v, sem):
    wid = lax.axis_index("s") * NC + lax.axis_index("c")
    lane = lax.iota(jnp.int32, L)

    pltpu.sync_copy(sid_hbm.at[pl.ds(wid * IPW, IPW)], sidf.at[pl.ds(0, IPW)])
    pltpu.sync_copy(rs_hbm.at[pl.ds(wid * RSW, RSW)], rsf.at[pl.ds(0, RSW)])
    sidf[pl.ds(IPW, L)] = jnp.zeros((L,), jnp.int32)

    ncols = rsf[pl.ds(RSW - L, L)][L - 1]
    nbatch = (ncols + NSLOT - 1) // NSLOT

    def batch_body(q, carry):
        runs = []
        for s in range(NSLOT):
            f = q * NSLOT + s
            rs0 = _sread(rsf, f)
            rs1 = _sread(rsf, f + 1)
            runs.append((rs0, rs1))
            col = _sread(sidf, rs0) // TILE_W
            start = pl.multiple_of(col * TILE_W, TILE_W)
            pltpu.async_copy(tab_hbm.at[:, pl.ds(start, TILE_W)],
                             blk_v.at[s], sem)
        for s in range(NSLOT):
            pltpu.make_async_copy(tab_hbm.at[:, pl.ds(0, TILE_W)],
                                  blk_v.at[s], sem).wait()
        for s in range(NSLOT):
            rs0, rs1 = runs[s]

            def id_body(i, carry2, s=s):
                sid = _sread(sidf, i)
                jv = jnp.full((L,), lax.rem(sid, TILE_W), jnp.int32)
                sv = jnp.full((L,), s, jnp.int32)
                rowv = jnp.full((L,), i, jnp.int32)
                for k in range(D // L):
                    val = plsc.load_gather(blk_v, [sv, lane + k * L, jv])
                    plsc.store_scatter(outbox_v, [rowv, lane + k * L], val)
                return carry2

            lax.fori_loop(rs0, rs1, id_body, 0)
        return carry

    lax.fori_loop(0, nbatch, batch_body, 0)

    pltpu.sync_copy(outbox_v, gath_hbm.at[pl.ds(wid * IPW, IPW), :])


def _k2_body(dpos_hbm, rpos_hbm, dgath_hbm, rgath_hbm, out_hbm,
             didx_v, ridx_v, drows_v, rrows_v, out_v, sem):
    wid = lax.axis_index("s") * NC + lax.axis_index("c")
    base = wid * BPW
    lane = lax.iota(jnp.int32, L)

    pltpu.sync_copy(dpos_hbm.at[wid], didx_v)
    pltpu.sync_copy(rpos_hbm.at[wid], ridx_v)

    for h in range(2):
        copies = []
        for c in range(2):
            dst = pl.ds(c * 128, 128)
            copies.append(pltpu.async_copy(
                dgath_hbm.at[didx_v.at[2 * h + c]], drows_v.at[dst], sem))
            copies.append(pltpu.async_copy(
                rgath_hbm.at[ridx_v.at[2 * h + c]], rrows_v.at[dst], sem))
        for cp in copies:
            cp.wait()
        for m in range(256 // L):
            rowv = m * L + lane
            acc = jnp.zeros((L,), jnp.float32)
            for d in range(D):
                dv = jnp.full((L,), d, jnp.int32)
                a = plsc.load_gather(drows_v, [rowv, dv])
                b = plsc.load_gather(rrows_v, [rowv, dv])
                acc = acc + a * b
            out_v[pl.ds(h * 256 + m * L, L)] = acc

    pltpu.sync_copy(out_v, out_hbm.at[pl.ds(base, BPW)])


def _sched(ids):
    ids = ids.astype(jnp.int32)
    order = jnp.argsort(ids).astype(jnp.int32)
    sid = jnp.take(ids, order)
    pos = jnp.argsort(order).astype(jnp.int32)
    colsw = (sid // TILE_W).reshape(NW, IPW)
    first = jnp.concatenate(
        [jnp.ones((NW, 1), jnp.bool_), colsw[:, 1:] != colsw[:, :-1]], axis=1)
    fno = jnp.cumsum(first.astype(jnp.int32), axis=1) - 1
    ncols = fno[:, -1] + 1
    frange = jnp.arange(RSW, dtype=jnp.int32)
    runstart = jnp.sum(
        fno[:, None, :] < frange[None, :, None], axis=-1, dtype=jnp.int32)
    runstart = runstart.at[:, RSW - 1].set(ncols)
    return (sid.reshape(NW * IPW), runstart.reshape(NW * RSW), pos)


def kernel(donor_ids, receiver_ids, donor_table, receiver_table):
    dsid, drs, dpos = _sched(donor_ids)
    rsid, rrs, rpos = _sched(receiver_ids)

    mesh = plsc.VectorSubcoreMesh(core_axis_name="c", subcore_axis_name="s",
                                  num_cores=NC, num_subcores=NS)
    params = pltpu.CompilerParams(needs_layout_passes=False)

    k1 = pl.kernel(
        _k1_body,
        out_type=jax.ShapeDtypeStruct((B, 128), jnp.float32),
        mesh=mesh,
        compiler_params=params,
        scratch_types=[
            pltpu.VMEM((IPW + L,), jnp.int32),
            pltpu.VMEM((RSW + L,), jnp.int32),
            pltpu.VMEM((NSLOT, D, TILE_W), jnp.float32),
            pltpu.VMEM((IPW, 128), jnp.float32),
            pltpu.SemaphoreType.DMA,
        ],
    )
    dgath = k1(dsid, drs, donor_table.T)
    rgath = k1(rsid, rrs, receiver_table.T)

    k2 = pl.kernel(
        _k2_body,
        out_type=jax.ShapeDtypeStruct((B,), jnp.float32),
        mesh=mesh,
        compiler_params=params,
        scratch_types=[
            pltpu.VMEM((4, 128), jnp.int32),
            pltpu.VMEM((4, 128), jnp.int32),
            pltpu.VMEM((256, 128), jnp.float32),
            pltpu.VMEM((256, 128), jnp.float32),
            pltpu.VMEM((BPW,), jnp.float32),
            pltpu.SemaphoreType.DMA,
        ],
    )
    return k2(dpos.reshape(NW, 4, 128), rpos.reshape(NW, 4, 128),
              dgath, rgath)


# speedup vs baseline: 1.5343x; 1.0085x over previous
import jax
import jax.numpy as jnp
from jax import lax
from jax.experimental import pallas as pl
from jax.experimental.pallas import tpu as pltpu
from jax.experimental.pallas import tpu_sc as plsc

B = 16384
D = 64
NC = 2
NS = 16
L = 16
NW = NC * NS
BPW = B // NW
IPW = B // NW
TILE_W = 128
NSLOT = 7
RSW = 5 * 128


def _sread(flat, i):
    return flat[pl.ds(i, L)][0]


def _k1_body(sid_hbm, rs_hbm, tab_hbm, gath_hbm,
             sidf, rsf, blk_v, outbox_v, sem):
    wid = lax.axis_index("s") * NC + lax.axis_index("c")
    lane = lax.iota(jnp.int32, L)

    pltpu.sync_copy(sid_hbm.at[pl.ds(wid * IPW, IPW)], sidf.at[pl.ds(0, IPW)])
    pltpu.sync_copy(rs_hbm.at[pl.ds(wid * RSW, RSW)], rsf.at[pl.ds(0, RSW)])
    sidf[pl.ds(IPW, L)] = jnp.zeros((L,), jnp.int32)

    ncols = rsf[pl.ds(RSW - L, L)][L - 1]
    nbatch = (ncols + NSLOT - 1) // NSLOT

    def batch_body(q, carry):
        runs = []
        for s in range(NSLOT):
            f = q * NSLOT + s
            rs0 = _sread(rsf, f)
            rs1 = _sread(rsf, f + 1)
            runs.append((rs0, rs1))
            col = _sread(sidf, rs0) // TILE_W
            start = pl.multiple_of(col * TILE_W, TILE_W)
            pltpu.async_copy(tab_hbm.at[:, pl.ds(start, TILE_W)],
                             blk_v.at[s], sem)
        for s in range(NSLOT):
            pltpu.make_async_copy(tab_hbm.at[:, pl.ds(0, TILE_W)],
                                  blk_v.at[s], sem).wait()
        for s in range(NSLOT):
            rs0, rs1 = runs[s]

            def id_body(i, carry2, s=s):
                sid = _sread(sidf, i)
                jv = jnp.full((L,), lax.rem(sid, TILE_W), jnp.int32)
                sv = jnp.full((L,), s, jnp.int32)
                rowv = jnp.full((L,), i, jnp.int32)
                for k in range(D // L):
                    val = plsc.load_gather(blk_v, [sv, lane + k * L, jv])
                    plsc.store_scatter(outbox_v, [rowv, lane + k * L], val)
                return carry2

            lax.fori_loop(rs0, rs1, id_body, 0)
        return carry

    lax.fori_loop(0, nbatch, batch_body, 0)

    pltpu.sync_copy(outbox_v, gath_hbm.at[pl.ds(wid * IPW, IPW), :])


def _k2_body(dpos_hbm, rpos_hbm, dgath_hbm, rgath_hbm, out_hbm,
             didx_v, ridx_v, drows_v, rrows_v, out_v, sem):
    wid = lax.axis_index("s") * NC + lax.axis_index("c")
    base = wid * BPW
    lane = lax.iota(jnp.int32, L)

    pltpu.sync_copy(dpos_hbm.at[wid], didx_v)
    pltpu.sync_copy(rpos_hbm.at[wid], ridx_v)

    for h in range(2):
        copies = []
        for c in range(2):
            dst = pl.ds(c * 128, 128)
            copies.append(pltpu.async_copy(
                dgath_hbm.at[didx_v.at[2 * h + c]], drows_v.at[dst], sem))
            copies.append(pltpu.async_copy(
                rgath_hbm.at[ridx_v.at[2 * h + c]], rrows_v.at[dst], sem))
        for cp in copies:
            cp.wait()
        for m in range(256 // L):
            rowv = m * L + lane
            acc = jnp.zeros((L,), jnp.float32)
            for d in range(D):
                dv = jnp.full((L,), d, jnp.int32)
                a = plsc.load_gather(drows_v, [rowv, dv])
                b = plsc.load_gather(rrows_v, [rowv, dv])
                acc = acc + a * b
            out_v[pl.ds(h * 256 + m * L, L)] = acc

    pltpu.sync_copy(out_v, out_hbm.at[pl.ds(base, BPW)])


def _sched(ids):
    ids = ids.astype(jnp.int32)
    order = jnp.argsort(ids).astype(jnp.int32)
    sid = jnp.take(ids, order)
    pos = jnp.argsort(order).astype(jnp.int32)
    colsw = (sid // TILE_W).reshape(NW, IPW)
    first = jnp.concatenate(
        [jnp.ones((NW, 1), jnp.bool_), colsw[:, 1:] != colsw[:, :-1]], axis=1)
    fno = jnp.cumsum(first.astype(jnp.int32), axis=1) - 1
    ncols = fno[:, -1] + 1
    frange = jnp.arange(RSW, dtype=jnp.int32)
    runstart = jnp.sum(
        fno[:, None, :] < frange[None, :, None], axis=-1, dtype=jnp.int32)
    runstart = runstart.at[:, RSW - 1].set(ncols)
    return (sid.reshape(NW * IPW), runstart.reshape(NW * RSW), pos)


def kernel(donor_ids, receiver_ids, donor_table, receiver_table):
    dsid, drs, dpos = _sched(donor_ids)
    rsid, rrs, rpos = _sched(receiver_ids)

    mesh = plsc.VectorSubcoreMesh(core_axis_name="c", subcore_axis_name="s",
                                  num_cores=NC, num_subcores=NS)
    params = pltpu.CompilerParams(needs_layout_passes=False)

    k1 = pl.kernel(
        _k1_body,
        out_type=jax.ShapeDtypeStruct((B, 128), jnp.float32),
        mesh=mesh,
        compiler_params=params,
        scratch_types=[
            pltpu.VMEM((IPW + L,), jnp.int32),
            pltpu.VMEM((RSW + L,), jnp.int32),
            pltpu.VMEM((NSLOT, D, TILE_W), jnp.float32),
            pltpu.VMEM((IPW, 128), jnp.float32),
            pltpu.SemaphoreType.DMA,
        ],
    )
    dgath = k1(dsid, drs, donor_table.T)
    rgath = k1(rsid, rrs, receiver_table.T)

    k2 = pl.kernel(
        _k2_body,
        out_type=jax.ShapeDtypeStruct((B,), jnp.float32),
        mesh=mesh,
        compiler_params=params,
        scratch_types=[
            pltpu.VMEM((4, 128), jnp.int32),
            pltpu.VMEM((4, 128), jnp.int32),
            pltpu.VMEM((256, 128), jnp.float32),
            pltpu.VMEM((256, 128), jnp.float32),
            pltpu.VMEM((BPW,), jnp.float32),
            pltpu.SemaphoreType.DMA,
        ],
    )
    return k2(dpos.reshape(NW, 4, 128), rpos.reshape(NW, 4, 128),
              dgath, rgath)
